# baseline (speedup 1.0000x reference)
import functools

import jax
import jax.numpy as jnp
import numpy as np
from jax import lax
from jax.experimental import pallas as pl
from jax.experimental.pallas import tpu as pltpu
import jax.experimental.pallas.tpu_sc as plsc

_PATCH_PERCENTAGE = 0.5
_THRESHOLD = 0.3
_GAUSSIAN_STD = 0.25


def _weighted_scores(scores, line_drawing):
    B, _, H, W = line_drawing.shape
    y_coords = jnp.linspace(0.0, 1.0, H).reshape(1, 1, H, 1)
    x_coords = jnp.linspace(0.0, 1.0, W).reshape(1, 1, 1, W)
    total_mass = line_drawing.sum(axis=(2, 3)) + 1e-06
    sum_y = (line_drawing * y_coords).sum(axis=(2, 3))
    sum_x = (line_drawing * x_coords).sum(axis=(2, 3))
    centers = jnp.stack([(sum_y / total_mass)[:, 0], (sum_x / total_mass)[:, 0]],
                        axis=1)
    N = scores.shape[1]
    side = int(np.sqrt(N))
    y_patch = jnp.linspace(0.0, 1.0, side)
    x_patch = jnp.linspace(0.0, 1.0, side)
    grid_y, grid_x = jnp.meshgrid(y_patch, x_patch, indexing='ij')
    grid_coords = jnp.stack([grid_y.flatten(), grid_x.flatten()], axis=1)
    distances_sq = ((grid_coords[None, :, :] - centers[:, None, :]) ** 2).sum(-1)
    gaussian = jnp.exp(-distances_sq / (2.0 * _GAUSSIAN_STD ** 2))
    return scores * gaussian


def _select_body(tri_ref, tribf_ref, ws_ref, idx_ref, gidx_ref):
    b = pl.program_id(0)
    ws = ws_ref[0, 0, :]
    N = ws.shape[0]
    K = idx_ref.shape[-1]
    tri = tri_ref[...]
    wsi = ws.reshape(N, 1)
    wsj = ws.reshape(1, N)
    fone = jnp.float32(1.0)
    fzero = jnp.float32(0.0)
    gtf = jnp.where(wsj > wsi, fone, fzero)
    M = jnp.where(wsj == wsi, tri, gtf)
    a_colf = jnp.where(wsi > _THRESHOLD, fone, fzero)
    rhs = jnp.concatenate([jnp.ones((N, 1), jnp.float32),
                           fone - a_colf, a_colf], axis=1)
    s_m = jax.lax.dot(M, rhs, preferred_element_type=jnp.float32)
    rank_all = s_m[:, 0]
    rank_below = s_m[:, 1]
    rank_above = jax.lax.dot(tribf_ref[...], a_colf.astype(jnp.bfloat16),
                             preferred_element_type=jnp.float32)[:, 0]
    a = a_colf[:, 0] > 0.5
    n_above = jnp.sum(a_colf)
    use_topk = (n_above >= K) | (n_above == 0)
    rank_mixed = jnp.where(a, rank_above, n_above + rank_below)
    rank = jnp.where(use_topk, rank_all, rank_mixed).astype(jnp.int32)
    rr = lax.broadcasted_iota(jnp.int32, (K, N), 0)
    e2 = jnp.where(rr == rank.reshape(1, N), fone, fzero)
    iota_f = lax.broadcasted_iota(jnp.int32, (N, 1), 0).astype(jnp.float32)
    hl = jax.lax.dot(e2, iota_f, preferred_element_type=jnp.float32)
    idx = hl[:, 0].astype(jnp.int32)
    idx_ref[0, 0, :] = idx
    gidx_ref[0, 0, :] = idx + b * N


def _select_indices(ws):
    B, N = ws.shape
    K = int(N * _PATCH_PERCENTAGE)
    ws3 = ws.reshape(B, 1, N)
    tri = np.tril(np.ones((N, N), np.float32), -1)
    idx, gidx = pl.pallas_call(
        _select_body,
        grid=(B,),
        in_specs=[pl.BlockSpec((N, N), lambda b: (0, 0)),
                  pl.BlockSpec((N, N), lambda b: (0, 0)),
                  pl.BlockSpec((1, 1, N), lambda b: (b, 0, 0))],
        out_specs=[pl.BlockSpec((1, 1, K), lambda b: (b, 0, 0))] * 2,
        out_shape=[jax.ShapeDtypeStruct((B, 1, K), jnp.int32)] * 2,
        compiler_params=pltpu.CompilerParams(
            dimension_semantics=("parallel",)),
    )(jnp.asarray(tri), jnp.asarray(tri, jnp.bfloat16), ws3)
    return idx.reshape(B * K), gidx.reshape(B * K)


def _make_gather(rows, D, NC, NS, CH):
    NW = NC * NS
    rpw = rows // NW
    nch = rpw // CH
    mesh = plsc.VectorSubcoreMesh(core_axis_name="c", subcore_axis_name="s")

    @functools.partial(
        pl.kernel,
        mesh=mesh,
        out_type=jax.ShapeDtypeStruct((rows, D), jnp.float32),
        scratch_types=[
            pltpu.VMEM((rpw,), jnp.int32),
            pltpu.VMEM((rpw,), jnp.int32),
            pltpu.VMEM((CH, D), jnp.float32),
            pltpu.VMEM((CH, D), jnp.float32),
            pltpu.VMEM((CH, D), jnp.float32),
            pltpu.VMEM((CH, D), jnp.float32),
            pltpu.SemaphoreType.DMA,
            pltpu.SemaphoreType.DMA,
            pltpu.SemaphoreType.DMA,
            pltpu.SemaphoreType.DMA,
        ],
    )
    def gather(magno_hbm, pos_hbm, gidx_hbm, pidx_hbm, out_hbm,
               gidx_v, pidx_v, bm_a, bp_a, bm_b, bp_b,
               sm_a, sp_a, sm_b, sp_b):
        wid = lax.axis_index("s") * NC + lax.axis_index("c")
        base = wid * rpw
        pltpu.sync_copy(gidx_hbm.at[pl.ds(base, rpw)], gidx_v)
        pltpu.sync_copy(pidx_hbm.at[pl.ds(base, rpw)], pidx_v)

        def issue(c, bm, bp, sm, sp):
            pltpu.async_copy(magno_hbm.at[gidx_v.at[pl.ds(c * CH, CH)]], bm, sm)
            pltpu.async_copy(pos_hbm.at[pidx_v.at[pl.ds(c * CH, CH)]], bp, sp)

        def finish(c, bm, bp, sm, sp):
            pltpu.make_async_copy(
                magno_hbm.at[gidx_v.at[pl.ds(c * CH, CH)]], bm, sm).wait()
            pltpu.make_async_copy(
                pos_hbm.at[pidx_v.at[pl.ds(c * CH, CH)]], bp, sp).wait()

            def row(r, carry):
                for cc in range(D // 16):
                    s = cc * 16
                    bm[r, pl.ds(s, 16)] = bm[r, pl.ds(s, 16)] + bp[r, pl.ds(s, 16)]
                return carry

            lax.fori_loop(0, CH, row, 0)
            pltpu.sync_copy(bm, out_hbm.at[pl.ds(base + c * CH, CH)])

        issue(0, bm_a, bp_a, sm_a, sp_a)

        def body(i, carry):
            c0 = 2 * i
            issue(c0 + 1, bm_b, bp_b, sm_b, sp_b)
            finish(c0, bm_a, bp_a, sm_a, sp_a)
            issue(c0 + 2, bm_a, bp_a, sm_a, sp_a)
            finish(c0 + 1, bm_b, bp_b, sm_b, sp_b)
            return carry

        lax.fori_loop(0, nch // 2 - 1, body, 0)
        c0 = nch - 2
        issue(c0 + 1, bm_b, bp_b, sm_b, sp_b)
        finish(c0, bm_a, bp_a, sm_a, sp_a)
        finish(c0 + 1, bm_b, bp_b, sm_b, sp_b)

    return gather


def kernel(magno_patches, vit_positional_embedding, scores, line_drawing):
    B, N, D = magno_patches.shape
    K = int(N * _PATCH_PERCENTAGE)
    ws = _weighted_scores(scores, line_drawing)
    pidx, gidx = _select_indices(ws)
    magno_flat = magno_patches.reshape(B * N, D)
    pos = vit_positional_embedding[0, 1:, :]
    info = plsc.get_sparse_core_info()
    gather = _make_gather(B * K, D, info.num_cores, info.num_subcores, CH=16)
    out = gather(magno_flat, pos, gidx, pidx)
    return out.reshape(B, K, D)


# speedup vs baseline: 1.1018x; 1.1018x over previous
import functools

import jax
import jax.numpy as jnp
import numpy as np
from jax import lax
from jax.experimental import pallas as pl
from jax.experimental.pallas import tpu as pltpu
import jax.experimental.pallas.tpu_sc as plsc

_PATCH_PERCENTAGE = 0.5
_THRESHOLD = 0.3
_GAUSSIAN_STD = 0.25


def _weighted_scores(scores, line_drawing):
    B, _, H, W = line_drawing.shape
    y_coords = jnp.linspace(0.0, 1.0, H).reshape(1, 1, H, 1)
    x_coords = jnp.linspace(0.0, 1.0, W).reshape(1, 1, 1, W)
    total_mass = line_drawing.sum(axis=(2, 3)) + 1e-06
    sum_y = (line_drawing * y_coords).sum(axis=(2, 3))
    sum_x = (line_drawing * x_coords).sum(axis=(2, 3))
    centers = jnp.stack([(sum_y / total_mass)[:, 0], (sum_x / total_mass)[:, 0]],
                        axis=1)
    N = scores.shape[1]
    side = int(np.sqrt(N))
    y_patch = jnp.linspace(0.0, 1.0, side)
    x_patch = jnp.linspace(0.0, 1.0, side)
    grid_y, grid_x = jnp.meshgrid(y_patch, x_patch, indexing='ij')
    grid_coords = jnp.stack([grid_y.flatten(), grid_x.flatten()], axis=1)
    distances_sq = ((grid_coords[None, :, :] - centers[:, None, :]) ** 2).sum(-1)
    gaussian = jnp.exp(-distances_sq / (2.0 * _GAUSSIAN_STD ** 2))
    return scores * gaussian


def _select_body(tri_hbm, ws_ref, idx_ref, gidx_ref, tri_vmem, sem):
    b = pl.program_id(0)
    ws = ws_ref[0, 0, :]
    N = ws.shape[0]
    K = idx_ref.shape[-1]

    @pl.when(b == 0)
    def _():
        cp = pltpu.make_async_copy(tri_hbm, tri_vmem, sem)
        cp.start()
        cp.wait()

    tri = tri_vmem[...]
    wsi = ws.reshape(N, 1)
    wsj = ws.reshape(1, N)
    fone = jnp.float32(1.0)
    fzero = jnp.float32(0.0)
    gtf = jnp.where(wsj > wsi, fone, fzero)
    M = jnp.where(wsj == wsi, tri, gtf)
    a_colf = jnp.where(wsi > _THRESHOLD, fone, fzero)
    rhs = jnp.concatenate([jnp.ones((N, 1), jnp.float32),
                           fone - a_colf, a_colf], axis=1)
    s_m = jax.lax.dot(M, rhs, preferred_element_type=jnp.float32)
    rank_all = s_m[:, 0]
    rank_below = s_m[:, 1]
    rank_above = jax.lax.dot(tri, a_colf,
                             preferred_element_type=jnp.float32)[:, 0]
    a = a_colf[:, 0] > 0.5
    n_above = jnp.sum(a_colf)
    use_topk = (n_above >= K) | (n_above == 0)
    rank_mixed = jnp.where(a, rank_above, n_above + rank_below)
    rank = jnp.where(use_topk, rank_all, rank_mixed).astype(jnp.int32)
    rr = lax.broadcasted_iota(jnp.int32, (K, N), 0)
    e2 = jnp.where(rr == rank.reshape(1, N), fone, fzero)
    iota_i = lax.broadcasted_iota(jnp.int32, (N, 1), 0)
    hi_lo = jnp.concatenate([(iota_i >> 5).astype(jnp.float32),
                             (iota_i & 31).astype(jnp.float32)], axis=1)
    hl = jax.lax.dot(e2, hi_lo, preferred_element_type=jnp.float32)
    idx = hl[:, 0].astype(jnp.int32) * 32 + hl[:, 1].astype(jnp.int32)
    idx_ref[0, 0, :] = idx
    gidx_ref[0, 0, :] = idx + b * N


def _select_indices(ws):
    B, N = ws.shape
    K = int(N * _PATCH_PERCENTAGE)
    ws3 = ws.reshape(B, 1, N)
    tri = np.tril(np.ones((N, N), np.float32), -1)
    idx, gidx = pl.pallas_call(
        _select_body,
        grid=(B,),
        in_specs=[pl.BlockSpec(memory_space=pltpu.MemorySpace.HBM),
                  pl.BlockSpec((1, 1, N), lambda b: (b, 0, 0))],
        out_specs=[pl.BlockSpec((1, 1, K), lambda b: (b, 0, 0))] * 2,
        out_shape=[jax.ShapeDtypeStruct((B, 1, K), jnp.int32)] * 2,
        scratch_shapes=[pltpu.VMEM((N, N), jnp.float32),
                        pltpu.SemaphoreType.DMA],
        compiler_params=pltpu.CompilerParams(
            dimension_semantics=("arbitrary",)),
    )(jnp.asarray(tri), ws3)
    return idx.reshape(B * K), gidx.reshape(B * K)


def _make_gather(rows, D, NC, NS, CH):
    NW = NC * NS
    rpw = rows // NW
    nch = rpw // CH
    mesh = plsc.VectorSubcoreMesh(core_axis_name="c", subcore_axis_name="s")

    @functools.partial(
        pl.kernel,
        mesh=mesh,
        out_type=jax.ShapeDtypeStruct((rows, D), jnp.float32),
        scratch_types=[
            pltpu.VMEM((rpw,), jnp.int32),
            pltpu.VMEM((rpw,), jnp.int32),
            pltpu.VMEM((CH, D), jnp.float32),
            pltpu.VMEM((CH, D), jnp.float32),
            pltpu.VMEM((CH, D), jnp.float32),
            pltpu.VMEM((CH, D), jnp.float32),
            pltpu.SemaphoreType.DMA,
            pltpu.SemaphoreType.DMA,
            pltpu.SemaphoreType.DMA,
            pltpu.SemaphoreType.DMA,
        ],
    )
    def gather(magno_hbm, pos_hbm, gidx_hbm, pidx_hbm, out_hbm,
               gidx_v, pidx_v, bm_a, bp_a, bm_b, bp_b,
               sm_a, sp_a, sm_b, sp_b):
        wid = lax.axis_index("s") * NC + lax.axis_index("c")
        base = wid * rpw
        pltpu.sync_copy(gidx_hbm.at[pl.ds(base, rpw)], gidx_v)
        pltpu.sync_copy(pidx_hbm.at[pl.ds(base, rpw)], pidx_v)

        def issue(c, bm, bp, sm, sp):
            pltpu.async_copy(magno_hbm.at[gidx_v.at[pl.ds(c * CH, CH)]], bm, sm)
            pltpu.async_copy(pos_hbm.at[pidx_v.at[pl.ds(c * CH, CH)]], bp, sp)

        def finish(c, bm, bp, sm, sp):
            pltpu.make_async_copy(
                magno_hbm.at[gidx_v.at[pl.ds(c * CH, CH)]], bm, sm).wait()
            pltpu.make_async_copy(
                pos_hbm.at[pidx_v.at[pl.ds(c * CH, CH)]], bp, sp).wait()

            def row(r, carry):
                for cc in range(D // 16):
                    s = cc * 16
                    bm[r, pl.ds(s, 16)] = bm[r, pl.ds(s, 16)] + bp[r, pl.ds(s, 16)]
                return carry

            lax.fori_loop(0, CH, row, 0)
            pltpu.sync_copy(bm, out_hbm.at[pl.ds(base + c * CH, CH)])

        issue(0, bm_a, bp_a, sm_a, sp_a)

        def body(i, carry):
            c0 = 2 * i
            issue(c0 + 1, bm_b, bp_b, sm_b, sp_b)
            finish(c0, bm_a, bp_a, sm_a, sp_a)
            issue(c0 + 2, bm_a, bp_a, sm_a, sp_a)
            finish(c0 + 1, bm_b, bp_b, sm_b, sp_b)
            return carry

        lax.fori_loop(0, nch // 2 - 1, body, 0)
        c0 = nch - 2
        issue(c0 + 1, bm_b, bp_b, sm_b, sp_b)
        finish(c0, bm_a, bp_a, sm_a, sp_a)
        finish(c0 + 1, bm_b, bp_b, sm_b, sp_b)

    return gather


def kernel(magno_patches, vit_positional_embedding, scores, line_drawing):
    B, N, D = magno_patches.shape
    K = int(N * _PATCH_PERCENTAGE)
    ws = _weighted_scores(scores, line_drawing)
    pidx, gidx = _select_indices(ws)
    magno_flat = magno_patches.reshape(B * N, D)
    pos = vit_positional_embedding[0, 1:, :]
    info = plsc.get_sparse_core_info()
    gather = _make_gather(B * K, D, info.num_cores, info.num_subcores, CH=16)
    out = gather(magno_flat, pos, gidx, pidx)
    return out.reshape(B, K, D)


# speedup vs baseline: 1.2301x; 1.1164x over previous
import functools

import jax
import jax.numpy as jnp
import numpy as np
from jax import lax
from jax.experimental import pallas as pl
from jax.experimental.pallas import tpu as pltpu
import jax.experimental.pallas.tpu_sc as plsc

_PATCH_PERCENTAGE = 0.5
_THRESHOLD = 0.3
_GAUSSIAN_STD = 0.25


def _weighted_scores(scores, line_drawing):
    B, _, H, W = line_drawing.shape
    y_coords = jnp.linspace(0.0, 1.0, H).reshape(1, 1, H, 1)
    x_coords = jnp.linspace(0.0, 1.0, W).reshape(1, 1, 1, W)
    total_mass = line_drawing.sum(axis=(2, 3)) + 1e-06
    sum_y = (line_drawing * y_coords).sum(axis=(2, 3))
    sum_x = (line_drawing * x_coords).sum(axis=(2, 3))
    centers = jnp.stack([(sum_y / total_mass)[:, 0], (sum_x / total_mass)[:, 0]],
                        axis=1)
    N = scores.shape[1]
    side = int(np.sqrt(N))
    y_patch = jnp.linspace(0.0, 1.0, side)
    x_patch = jnp.linspace(0.0, 1.0, side)
    grid_y, grid_x = jnp.meshgrid(y_patch, x_patch, indexing='ij')
    grid_coords = jnp.stack([grid_y.flatten(), grid_x.flatten()], axis=1)
    distances_sq = ((grid_coords[None, :, :] - centers[:, None, :]) ** 2).sum(-1)
    gaussian = jnp.exp(-distances_sq / (2.0 * _GAUSSIAN_STD ** 2))
    return scores * gaussian


def _select_body(tri_hbm, ws_ref, idx_ref, gidx_ref, tri_vmem, sem):
    b = pl.program_id(0)
    SPB = ws_ref.shape[0]
    N = ws_ref.shape[-1]
    K = idx_ref.shape[-1]

    @pl.when(b == 0)
    def _():
        cp = pltpu.make_async_copy(tri_hbm, tri_vmem, sem)
        cp.start()
        cp.wait()

    tri = tri_vmem[...]
    fone = jnp.float32(1.0)
    fzero = jnp.float32(0.0)
    for r in range(SPB):
        ws = ws_ref[r, 0, :]
        wsi = ws.reshape(N, 1)
        wsj = ws.reshape(1, N)
        gtf = jnp.where(wsj > wsi, fone, fzero)
        M = jnp.where(wsj == wsi, tri, gtf)
        a_colf = jnp.where(wsi > _THRESHOLD, fone, fzero)
        rhs = jnp.concatenate([jnp.ones((N, 1), jnp.float32),
                               fone - a_colf, a_colf], axis=1)
        s_m = jax.lax.dot(M, rhs, preferred_element_type=jnp.float32)
        rank_all = s_m[:, 0]
        rank_below = s_m[:, 1]
        rank_above = jax.lax.dot(tri, a_colf,
                                 preferred_element_type=jnp.float32)[:, 0]
        a = a_colf[:, 0] > 0.5
        n_above = jnp.sum(a_colf)
        use_topk = (n_above >= K) | (n_above == 0)
        rank_mixed = jnp.where(a, rank_above, n_above + rank_below)
        rank = jnp.where(use_topk, rank_all, rank_mixed).astype(jnp.int32)
        rr = lax.broadcasted_iota(jnp.int32, (K, N), 0)
        e2 = jnp.where(rr == rank.reshape(1, N), fone, fzero)
        iota_i = lax.broadcasted_iota(jnp.int32, (N, 1), 0)
        hi_lo = jnp.concatenate([(iota_i >> 5).astype(jnp.float32),
                                 (iota_i & 31).astype(jnp.float32)], axis=1)
        hl = jax.lax.dot(e2, hi_lo, preferred_element_type=jnp.float32)
        idx = hl[:, 0].astype(jnp.int32) * 32 + hl[:, 1].astype(jnp.int32)
        idx_ref[r, 0, :] = idx
        gidx_ref[r, 0, :] = idx + (b * SPB + r) * N


def _select_indices(ws):
    B, N = ws.shape
    K = int(N * _PATCH_PERCENTAGE)
    ws3 = ws.reshape(B, 1, N)
    tri = np.tril(np.ones((N, N), np.float32), -1)
    SPB = 2
    idx, gidx = pl.pallas_call(
        _select_body,
        grid=(B // SPB,),
        in_specs=[pl.BlockSpec(memory_space=pltpu.MemorySpace.HBM),
                  pl.BlockSpec((SPB, 1, N), lambda b: (b, 0, 0))],
        out_specs=[pl.BlockSpec((SPB, 1, K), lambda b: (b, 0, 0))] * 2,
        out_shape=[jax.ShapeDtypeStruct((B, 1, K), jnp.int32)] * 2,
        scratch_shapes=[pltpu.VMEM((N, N), jnp.float32),
                        pltpu.SemaphoreType.DMA],
        compiler_params=pltpu.CompilerParams(
            dimension_semantics=("arbitrary",)),
    )(jnp.asarray(tri), ws3)
    return idx.reshape(B * K), gidx.reshape(B * K)


def _make_gather(rows, D, NC, NS, CH):
    NW = NC * NS
    rpw = rows // NW
    nch = rpw // CH
    mesh = plsc.VectorSubcoreMesh(core_axis_name="c", subcore_axis_name="s")

    @functools.partial(
        pl.kernel,
        mesh=mesh,
        out_type=jax.ShapeDtypeStruct((rows, D), jnp.float32),
        scratch_types=[
            pltpu.VMEM((rpw,), jnp.int32),
            pltpu.VMEM((rpw,), jnp.int32),
            pltpu.VMEM((CH, D), jnp.float32),
            pltpu.VMEM((CH, D), jnp.float32),
            pltpu.VMEM((CH, D), jnp.float32),
            pltpu.VMEM((CH, D), jnp.float32),
            pltpu.SemaphoreType.DMA,
            pltpu.SemaphoreType.DMA,
            pltpu.SemaphoreType.DMA,
            pltpu.SemaphoreType.DMA,
        ],
    )
    def gather(magno_hbm, pos_hbm, gidx_hbm, pidx_hbm, out_hbm,
               gidx_v, pidx_v, bm_a, bp_a, bm_b, bp_b,
               sm_a, sp_a, sm_b, sp_b):
        wid = lax.axis_index("s") * NC + lax.axis_index("c")
        base = wid * rpw
        pltpu.sync_copy(gidx_hbm.at[pl.ds(base, rpw)], gidx_v)
        pltpu.sync_copy(pidx_hbm.at[pl.ds(base, rpw)], pidx_v)

        def issue(c, bm, bp, sm, sp):
            pltpu.async_copy(magno_hbm.at[gidx_v.at[pl.ds(c * CH, CH)]], bm, sm)
            pltpu.async_copy(pos_hbm.at[pidx_v.at[pl.ds(c * CH, CH)]], bp, sp)

        def finish(c, bm, bp, sm, sp):
            pltpu.make_async_copy(
                magno_hbm.at[gidx_v.at[pl.ds(c * CH, CH)]], bm, sm).wait()
            pltpu.make_async_copy(
                pos_hbm.at[pidx_v.at[pl.ds(c * CH, CH)]], bp, sp).wait()

            def row(r, carry):
                for cc in range(D // 16):
                    s = cc * 16
                    bm[r, pl.ds(s, 16)] = bm[r, pl.ds(s, 16)] + bp[r, pl.ds(s, 16)]
                return carry

            lax.fori_loop(0, CH, row, 0)
            pltpu.sync_copy(bm, out_hbm.at[pl.ds(base + c * CH, CH)])

        issue(0, bm_a, bp_a, sm_a, sp_a)

        def body(i, carry):
            c0 = 2 * i
            issue(c0 + 1, bm_b, bp_b, sm_b, sp_b)
            finish(c0, bm_a, bp_a, sm_a, sp_a)
            issue(c0 + 2, bm_a, bp_a, sm_a, sp_a)
            finish(c0 + 1, bm_b, bp_b, sm_b, sp_b)
            return carry

        lax.fori_loop(0, nch // 2 - 1, body, 0)
        c0 = nch - 2
        issue(c0 + 1, bm_b, bp_b, sm_b, sp_b)
        finish(c0, bm_a, bp_a, sm_a, sp_a)
        finish(c0 + 1, bm_b, bp_b, sm_b, sp_b)

    return gather


def kernel(magno_patches, vit_positional_embedding, scores, line_drawing):
    B, N, D = magno_patches.shape
    K = int(N * _PATCH_PERCENTAGE)
    ws = _weighted_scores(scores, line_drawing)
    pidx, gidx = _select_indices(ws)
    magno_flat = magno_patches.reshape(B * N, D)
    pos = vit_positional_embedding[0, 1:, :]
    info = plsc.get_sparse_core_info()
    gather = _make_gather(B * K, D, info.num_cores, info.num_subcores, CH=16)
    out = gather(magno_flat, pos, gidx, pidx)
    return out.reshape(B, K, D)


# speedup vs baseline: 1.2399x; 1.0080x over previous
import functools

import jax
import jax.numpy as jnp
import numpy as np
from jax import lax
from jax.experimental import pallas as pl
from jax.experimental.pallas import tpu as pltpu
import jax.experimental.pallas.tpu_sc as plsc

_PATCH_PERCENTAGE = 0.5
_THRESHOLD = 0.3
_GAUSSIAN_STD = 0.25


def _weighted_scores(scores, line_drawing):
    B, _, H, W = line_drawing.shape
    y_coords = jnp.linspace(0.0, 1.0, H).reshape(1, 1, H, 1)
    x_coords = jnp.linspace(0.0, 1.0, W).reshape(1, 1, 1, W)
    total_mass = line_drawing.sum(axis=(2, 3)) + 1e-06
    sum_y = (line_drawing * y_coords).sum(axis=(2, 3))
    sum_x = (line_drawing * x_coords).sum(axis=(2, 3))
    centers = jnp.stack([(sum_y / total_mass)[:, 0], (sum_x / total_mass)[:, 0]],
                        axis=1)
    N = scores.shape[1]
    side = int(np.sqrt(N))
    y_patch = jnp.linspace(0.0, 1.0, side)
    x_patch = jnp.linspace(0.0, 1.0, side)
    grid_y, grid_x = jnp.meshgrid(y_patch, x_patch, indexing='ij')
    grid_coords = jnp.stack([grid_y.flatten(), grid_x.flatten()], axis=1)
    distances_sq = ((grid_coords[None, :, :] - centers[:, None, :]) ** 2).sum(-1)
    gaussian = jnp.exp(-distances_sq / (2.0 * _GAUSSIAN_STD ** 2))
    return scores * gaussian


def _select_body(tri_hbm, ws_ref, idx_ref, gidx_ref, tri_vmem, sem):
    b = pl.program_id(0)
    SPB = ws_ref.shape[0]
    N = ws_ref.shape[-1]
    K = idx_ref.shape[-1]

    @pl.when(b == 0)
    def _():
        cp = pltpu.make_async_copy(tri_hbm, tri_vmem, sem)
        cp.start()
        cp.wait()

    tri = tri_vmem[...]
    fone = jnp.float32(1.0)
    fzero = jnp.float32(0.0)
    for r in range(SPB):
        ws = ws_ref[r, 0, :]
        wsi = ws.reshape(N, 1)
        wsj = ws.reshape(1, N)
        gtf = jnp.where(wsj > wsi, fone, fzero)
        M = jnp.where(wsj == wsi, tri, gtf)
        a_colf = jnp.where(wsi > _THRESHOLD, fone, fzero)
        rhs = jnp.concatenate([jnp.ones((N, 1), jnp.float32),
                               fone - a_colf, a_colf], axis=1)
        s_m = jax.lax.dot(M, rhs, preferred_element_type=jnp.float32)
        rank_all = s_m[:, 0]
        rank_below = s_m[:, 1]
        rank_above = jax.lax.dot(tri, a_colf,
                                 preferred_element_type=jnp.float32)[:, 0]
        a = a_colf[:, 0] > 0.5
        n_above = jnp.sum(a_colf)
        use_topk = (n_above >= K) | (n_above == 0)
        rank_mixed = jnp.where(a, rank_above, n_above + rank_below)
        rank = jnp.where(use_topk, rank_all, rank_mixed).astype(jnp.int32)
        rr = lax.broadcasted_iota(jnp.int32, (K, N), 0)
        e2 = jnp.where(rr == rank.reshape(1, N), fone, fzero)
        iota_i = lax.broadcasted_iota(jnp.int32, (N, 1), 0)
        hi_lo = jnp.concatenate([(iota_i >> 5).astype(jnp.float32),
                                 (iota_i & 31).astype(jnp.float32)], axis=1)
        hl = jax.lax.dot(e2, hi_lo, preferred_element_type=jnp.float32)
        idx = hl[:, 0].astype(jnp.int32) * 32 + hl[:, 1].astype(jnp.int32)
        idx_ref[r, 0, :] = idx
        gidx_ref[r, 0, :] = idx + (b * SPB + r) * N


def _select_indices(ws):
    B, N = ws.shape
    K = int(N * _PATCH_PERCENTAGE)
    ws3 = ws.reshape(B, 1, N)
    tri = np.tril(np.ones((N, N), np.float32), -1)
    SPB = 4
    idx, gidx = pl.pallas_call(
        _select_body,
        grid=(B // SPB,),
        in_specs=[pl.BlockSpec(memory_space=pltpu.MemorySpace.HBM),
                  pl.BlockSpec((SPB, 1, N), lambda b: (b, 0, 0))],
        out_specs=[pl.BlockSpec((SPB, 1, K), lambda b: (b, 0, 0))] * 2,
        out_shape=[jax.ShapeDtypeStruct((B, 1, K), jnp.int32)] * 2,
        scratch_shapes=[pltpu.VMEM((N, N), jnp.float32),
                        pltpu.SemaphoreType.DMA],
        compiler_params=pltpu.CompilerParams(
            dimension_semantics=("arbitrary",)),
    )(jnp.asarray(tri), ws3)
    return idx.reshape(B * K), gidx.reshape(B * K)


def _make_gather(rows, D, NC, NS, CH):
    NW = NC * NS
    rpw = rows // NW
    nch = rpw // CH
    mesh = plsc.VectorSubcoreMesh(core_axis_name="c", subcore_axis_name="s")

    @functools.partial(
        pl.kernel,
        mesh=mesh,
        out_type=jax.ShapeDtypeStruct((rows, D), jnp.float32),
        scratch_types=[
            pltpu.VMEM((rpw,), jnp.int32),
            pltpu.VMEM((rpw,), jnp.int32),
            pltpu.VMEM((CH, D), jnp.float32),
            pltpu.VMEM((CH, D), jnp.float32),
            pltpu.VMEM((CH, D), jnp.float32),
            pltpu.VMEM((CH, D), jnp.float32),
            pltpu.SemaphoreType.DMA,
            pltpu.SemaphoreType.DMA,
            pltpu.SemaphoreType.DMA,
            pltpu.SemaphoreType.DMA,
        ],
    )
    def gather(magno_hbm, pos_hbm, gidx_hbm, pidx_hbm, out_hbm,
               gidx_v, pidx_v, bm_a, bp_a, bm_b, bp_b,
               sm_a, sp_a, sm_b, sp_b):
        wid = lax.axis_index("s") * NC + lax.axis_index("c")
        base = wid * rpw
        pltpu.sync_copy(gidx_hbm.at[pl.ds(base, rpw)], gidx_v)
        pltpu.sync_copy(pidx_hbm.at[pl.ds(base, rpw)], pidx_v)

        def issue(c, bm, bp, sm, sp):
            pltpu.async_copy(magno_hbm.at[gidx_v.at[pl.ds(c * CH, CH)]], bm, sm)
            pltpu.async_copy(pos_hbm.at[pidx_v.at[pl.ds(c * CH, CH)]], bp, sp)

        def finish(c, bm, bp, sm, sp):
            pltpu.make_async_copy(
                magno_hbm.at[gidx_v.at[pl.ds(c * CH, CH)]], bm, sm).wait()
            pltpu.make_async_copy(
                pos_hbm.at[pidx_v.at[pl.ds(c * CH, CH)]], bp, sp).wait()

            def row(r, carry):
                for cc in range(D // 16):
                    s = cc * 16
                    bm[r, pl.ds(s, 16)] = bm[r, pl.ds(s, 16)] + bp[r, pl.ds(s, 16)]
                return carry

            lax.fori_loop(0, CH, row, 0)
            pltpu.sync_copy(bm, out_hbm.at[pl.ds(base + c * CH, CH)])

        issue(0, bm_a, bp_a, sm_a, sp_a)

        def body(i, carry):
            c0 = 2 * i
            issue(c0 + 1, bm_b, bp_b, sm_b, sp_b)
            finish(c0, bm_a, bp_a, sm_a, sp_a)
            issue(c0 + 2, bm_a, bp_a, sm_a, sp_a)
            finish(c0 + 1, bm_b, bp_b, sm_b, sp_b)
            return carry

        lax.fori_loop(0, nch // 2 - 1, body, 0)
        c0 = nch - 2
        issue(c0 + 1, bm_b, bp_b, sm_b, sp_b)
        finish(c0, bm_a, bp_a, sm_a, sp_a)
        finish(c0 + 1, bm_b, bp_b, sm_b, sp_b)

    return gather


def kernel(magno_patches, vit_positional_embedding, scores, line_drawing):
    B, N, D = magno_patches.shape
    K = int(N * _PATCH_PERCENTAGE)
    ws = _weighted_scores(scores, line_drawing)
    pidx, gidx = _select_indices(ws)
    magno_flat = magno_patches.reshape(B * N, D)
    pos = vit_positional_embedding[0, 1:, :]
    info = plsc.get_sparse_core_info()
    gather = _make_gather(B * K, D, info.num_cores, info.num_subcores, CH=16)
    out = gather(magno_flat, pos, gidx, pidx)
    return out.reshape(B, K, D)
